# baseline (speedup 1.0000x reference)
import functools

import jax
import jax.numpy as jnp
import numpy as np
from jax import lax
from jax.experimental import pallas as pl
from jax.experimental.pallas import tpu as pltpu
from jax.experimental.pallas import tpu_sc as plsc

N_NODES = 10000
D_FEAT = 128
N_GRAPHS = 64
N_CLASSES = 10
H = 64
AH = 4
HD = H // AH
LAYERS = 3
BN_EPS = 1e-05
MAX_LEN = 512

_BLK = 2000
_PAD = N_NODES + MAX_LEN
NPAD = 10240
DUMMY = N_NODES

NW = 32
W_E = 10368
E_PAD = NW * W_E
CH = 576
NCH = W_E // CH
NT = CH // 16

_INV_SQRT1P = float(1.0 / np.sqrt(1.0 + BN_EPS))


def _gelu_exact(x):
    return 0.5 * x * (1.0 + lax.erf(x * float(1.0 / np.sqrt(2.0))))


def _pre_mlp_body(x_ref, w1_ref, b1_ref, w2_ref, b2_ref, o_ref):
    h = _gelu_exact(x_ref[...] @ w1_ref[...] + b1_ref[...])
    o_ref[...] = _gelu_exact(h @ w2_ref[...] + b2_ref[...])


def _pre_mlp(x, W1, b1, W2, b2):
    return pl.pallas_call(
        _pre_mlp_body,
        grid=(N_NODES // _BLK,),
        in_specs=[
            pl.BlockSpec((_BLK, D_FEAT), lambda i: (i, 0)),
            pl.BlockSpec((D_FEAT, 2 * H), lambda i: (0, 0)),
            pl.BlockSpec((1, 2 * H), lambda i: (0, 0)),
            pl.BlockSpec((2 * H, H), lambda i: (0, 0)),
            pl.BlockSpec((1, H), lambda i: (0, 0)),
        ],
        out_specs=pl.BlockSpec((_BLK, H), lambda i: (i, 0)),
        out_shape=jax.ShapeDtypeStruct((N_NODES, H), jnp.float32),
    )(x, W1, b1[None, :], W2, b2[None, :])


def _proj_body(h_ref, wl_ref, bl_ref, wr_ref, br_ref, xl_ref, xr_ref):
    h = h_ref[...]
    xl_ref[...] = h @ wl_ref[...] + bl_ref[...]
    xr_ref[...] = h @ wr_ref[...] + br_ref[...]


def _gat_proj(h_npad, Wl, bl, Wr, br):
    return pl.pallas_call(
        _proj_body,
        grid=(NPAD // 1280,),
        in_specs=[
            pl.BlockSpec((1280, H), lambda i: (i, 0)),
            pl.BlockSpec((H, H), lambda i: (0, 0)),
            pl.BlockSpec((1, H), lambda i: (0, 0)),
            pl.BlockSpec((H, H), lambda i: (0, 0)),
            pl.BlockSpec((1, H), lambda i: (0, 0)),
        ],
        out_specs=[
            pl.BlockSpec((1280, H), lambda i: (i, 0)),
            pl.BlockSpec((1280, H), lambda i: (i, 0)),
        ],
        out_shape=[jax.ShapeDtypeStruct((NPAD, H), jnp.float32),
                   jax.ShapeDtypeStruct((NPAD, H), jnp.float32)],
    )(h_npad, Wl, bl[None, :], Wr, br[None, :])


_AB = 4096
_ANB = E_PAD // _AB


def _alpha_body(s_ref, att_ref, a_ref, g_ref, m_ref):
    i = pl.program_id(0)
    s = s_ref[...]
    a = jnp.maximum(s, 0.2 * s) @ att_ref[...]
    a_ref[...] = a

    @pl.when(i == 0)
    def _():
        m_ref[0] = -jnp.inf

    m_ref[0] = jnp.maximum(m_ref[0], jnp.max(a))

    @pl.when(i == _ANB - 1)
    def _():
        g_ref[...] = jnp.full((1, 128), m_ref[0], jnp.float32)


def _tc_alpha(s_rows, att):
    alpha, g = pl.pallas_call(
        _alpha_body,
        grid=(_ANB,),
        in_specs=[
            pl.BlockSpec((_AB, H), lambda i: (i, 0)),
            pl.BlockSpec((H, 1), lambda i: (0, 0)),
        ],
        out_specs=[
            pl.BlockSpec((_AB, 1), lambda i: (i, 0)),
            pl.BlockSpec((1, 128), lambda i: (0, 0)),
        ],
        out_shape=[jax.ShapeDtypeStruct((E_PAD, 1), jnp.float32),
                   jax.ShapeDtypeStruct((1, 128), jnp.float32)],
        scratch_shapes=[pltpu.SMEM((1,), jnp.float32)],
    )(s_rows, att[:, None])
    return alpha.reshape(E_PAD), g.reshape(128)


_SC_MESH = plsc.VectorSubcoreMesh(core_axis_name="c", subcore_axis_name="s")
_SC_PARAMS = pltpu.CompilerParams(needs_layout_passes=False,
                                  use_tc_tiling_on_sc=False)


@functools.partial(
    pl.kernel, mesh=_SC_MESH,
    out_type=jax.ShapeDtypeStruct((E_PAD, H), jnp.float32),
    scratch_types=[pltpu.VMEM((CH,), jnp.int32),
                   pltpu.VMEM((CH,), jnp.int32),
                   pltpu.VMEM((CH, H), jnp.float32),
                   pltpu.SemaphoreType.DMA],
    compiler_params=_SC_PARAMS,
)
def _sc_stream(xl_hbm, xr_hbm, src_hbm, dst_hbm, s_hbm,
               src_v, dst_v, s_v, sem1):
    cc = lax.axis_index("c")
    ss = lax.axis_index("s")
    wid = ss * 2 + cc

    def chunk(j, _):
        base = wid * W_E + j * CH
        pltpu.sync_copy(src_hbm.at[pl.ds(base, CH)], src_v)
        pltpu.sync_copy(dst_hbm.at[pl.ds(base, CH)], dst_v)
        pltpu.async_copy(xr_hbm.at[dst_v], s_v, sem1).wait()
        pltpu.async_copy(xl_hbm.at[src_v], s_v, sem1, add=True).wait()
        pltpu.sync_copy(s_v, s_hbm.at[pl.ds(base, CH)])
        return 0

    lax.fori_loop(0, NCH, chunk, 0)


@functools.partial(
    pl.kernel, mesh=_SC_MESH,
    out_type=[jax.ShapeDtypeStruct((E_PAD,), jnp.float32),
              jax.ShapeDtypeStruct((NW, NPAD), jnp.float32)],
    scratch_types=[pltpu.VMEM((NPAD,), jnp.float32),
                   pltpu.VMEM((16,), jnp.float32),
                   pltpu.VMEM((CH,), jnp.int32),
                   pltpu.VMEM((CH,), jnp.float32),
                   pltpu.VMEM((CH,), jnp.float32)],
    compiler_params=_SC_PARAMS,
)
def _sc_den(alpha_hbm, dst_hbm, g_hbm,
            ex_hbm, dparts_hbm,
            den_v, g_v, dst_v, a_v, ex_v):
    cc = lax.axis_index("c")
    ss = lax.axis_index("s")
    wid = ss * 2 + cc
    zero16 = jnp.zeros((16,), jnp.float32)

    pltpu.sync_copy(g_hbm.at[pl.ds(0, 16)], g_v)
    gvec = g_v[...]

    def zbody(i, _):
        den_v[pl.ds(i * 16, 16)] = zero16
        return 0

    lax.fori_loop(0, NPAD // 16, zbody, 0)

    def chunk(j, _):
        base = wid * W_E + j * CH
        pltpu.sync_copy(dst_hbm.at[pl.ds(base, CH)], dst_v)
        pltpu.sync_copy(alpha_hbm.at[pl.ds(base, CH)], a_v)

        def group(t, _):
            dstv = dst_v[pl.ds(t * 16, 16)]
            ex = jnp.exp(a_v[pl.ds(t * 16, 16)] - gvec)
            ex_v[pl.ds(t * 16, 16)] = ex
            plsc.addupdate_scatter(den_v, [dstv], ex)
            return 0

        lax.fori_loop(0, NT, group, 0)
        pltpu.sync_copy(ex_v, ex_hbm.at[pl.ds(base, CH)])
        return 0

    lax.fori_loop(0, NCH, chunk, 0)
    pltpu.sync_copy(den_v, dparts_hbm.at[wid])


@functools.partial(
    pl.kernel, mesh=_SC_MESH,
    out_type=jax.ShapeDtypeStruct((2, NPAD, H), jnp.float32),
    scratch_types=[pltpu.VMEM((CH,), jnp.int32),
                   pltpu.VMEM((CH,), jnp.int32),
                   pltpu.VMEM((CH, H), jnp.float32),
                   pltpu.VMEM((CH,), jnp.float32),
                   pltpu.VMEM_SHARED((NPAD, H), jnp.float32),
                   pltpu.SemaphoreType.DMA],
    compiler_params=_SC_PARAMS,
)
def _sc_scatter(xl_hbm, src_hbm, dst_hbm, ex_hbm, zeros_hbm,
                out_hbm,
                src_v, dst_v, el_v, ex_v, accum_sh, sem1):
    cc = lax.axis_index("c")
    ss = lax.axis_index("s")
    wid = ss * 2 + cc
    lanes = lax.iota(jnp.int32, 16)

    pltpu.sync_copy(zeros_hbm, accum_sh.at[pl.ds(ss * 640, 640)])
    plsc.subcore_barrier()

    def chunk(j, _):
        base = wid * W_E + j * CH
        pltpu.sync_copy(src_hbm.at[pl.ds(base, CH)], src_v)
        pltpu.sync_copy(dst_hbm.at[pl.ds(base, CH)], dst_v)
        pltpu.sync_copy(ex_hbm.at[pl.ds(base, CH)], ex_v)
        pltpu.async_copy(xl_hbm.at[src_v], el_v, sem1).wait()

        def group(t, _):
            rows = t * 16 + lanes
            ex = ex_v[pl.ds(t * 16, 16)]
            for f in range(H):
                fv = jnp.full((16,), f, jnp.int32)
                val = plsc.load_gather(el_v, [rows, fv]) * ex
                plsc.store_scatter(el_v, [rows, fv], val)
            return 0

        lax.fori_loop(0, NT, group, 0)
        pltpu.sync_copy(el_v, accum_sh.at[dst_v], add=True)
        return 0

    lax.fori_loop(0, NCH, chunk, 0)
    plsc.subcore_barrier()
    pltpu.sync_copy(accum_sh.at[pl.ds(ss * 640, 640)],
                    out_hbm.at[cc, pl.ds(ss * 640, 640)])


def _mha_body(starts_ref, counts_ref, h_ref, wqkv_ref, bqkv_ref, wo_ref,
              bo_ref, out_ref):
    g = pl.program_id(0)
    start = starts_ref[g]
    count = counts_ref[g]
    hs = h_ref[pl.ds(start, MAX_LEN), :]
    qkv = hs @ wqkv_ref[...] + bqkv_ref[...]
    col = lax.broadcasted_iota(jnp.int32, (MAX_LEN, MAX_LEN), 1)
    kmask = col < count
    os = []
    for a in range(AH):
        q = qkv[:, a * HD:(a + 1) * HD]
        k = qkv[:, H + a * HD:H + (a + 1) * HD]
        v = qkv[:, 2 * H + a * HD:2 * H + (a + 1) * HD]
        s = lax.dot_general(q, k, (((1,), (1,)), ((), ()))) * (1.0 / np.sqrt(HD))
        s = jnp.where(kmask, s, -1e9)
        m = jnp.max(s, axis=-1, keepdims=True)
        p = jnp.exp(s - m)
        denom = jnp.sum(p, axis=-1, keepdims=True)
        os.append((p / denom) @ v)
    o = jnp.concatenate(os, axis=-1) @ wo_ref[...] + bo_ref[...]
    row = lax.broadcasted_iota(jnp.int32, (MAX_LEN, H), 0)
    cur = out_ref[pl.ds(start, MAX_LEN), :]
    out_ref[pl.ds(start, MAX_LEN), :] = jnp.where(row < count, o, cur)


def _mha_pallas(h_pad, starts, counts, Wqkv, bqkv, Wo, bo):
    out = pl.pallas_call(
        _mha_body,
        grid=(N_GRAPHS,),
        in_specs=[
            pl.BlockSpec(memory_space=pltpu.SMEM),
            pl.BlockSpec(memory_space=pltpu.SMEM),
            pl.BlockSpec((_PAD, H), lambda g: (0, 0)),
            pl.BlockSpec((H, 3 * H), lambda g: (0, 0)),
            pl.BlockSpec((1, 3 * H), lambda g: (0, 0)),
            pl.BlockSpec((H, H), lambda g: (0, 0)),
            pl.BlockSpec((1, H), lambda g: (0, 0)),
        ],
        out_specs=pl.BlockSpec((_PAD, H), lambda g: (0, 0)),
        out_shape=jax.ShapeDtypeStruct((_PAD, H), jnp.float32),
    )(starts, counts, h_pad, Wqkv, bqkv[None], Wo, bo[None])
    return out[:N_NODES]


def _combine_body(h_ref, p_ref, dp_ref, ones_ref, hat_ref,
                  gbias_ref, w1b_ref, b1b_ref, w2b_ref, b2b_ref,
                  mw1_ref, mb1_ref, mw2_ref, mb2_ref, w3b_ref, b3b_ref,
                  o_ref):
    h = h_ref[...]
    den = lax.dot_general(dp_ref[...], ones_ref[...],
                          (((0,), (0,)), ((), ())))
    gat = (p_ref[0] + p_ref[1])[:N_NODES] / den[:N_NODES]
    gat = gat + gbias_ref[...]
    h_local = (gat + h) * _INV_SQRT1P * w1b_ref[...] + b1b_ref[...]
    h_attn = (hat_ref[...] + h) * _INV_SQRT1P * w2b_ref[...] + b2b_ref[...]
    out = h_local + h_attn
    m = jnp.maximum(out @ mw1_ref[...] + mb1_ref[...], 0.0)
    m = m @ mw2_ref[...] + mb2_ref[...]
    out = (out + m) * _INV_SQRT1P * w3b_ref[...] + b3b_ref[...]
    o_ref[...] = jnp.maximum(out, 0.0)


def _combine(h, parts, dparts, h_attn, gbias, w1b, b1b, w2b, b2b,
             mw1, mb1, mw2, mb2, w3b, b3b):
    vec = lambda a: a[None, :]
    full2 = lambda shape: pl.BlockSpec(shape, lambda: tuple(0 for _ in shape))
    return pl.pallas_call(
        _combine_body,
        in_specs=[
            full2((N_NODES, H)),
            full2((2, NPAD, H)),
            full2((NW, NPAD)),
            full2((NW, 1)),
            full2((N_NODES, H)),
            full2((1, H)),
            full2((1, H)),
            full2((1, H)),
            full2((1, H)),
            full2((1, H)),
            full2((H, 2 * H)),
            full2((1, 2 * H)),
            full2((2 * H, H)),
            full2((1, H)),
            full2((1, H)),
            full2((1, H)),
        ],
        out_specs=full2((N_NODES, H)),
        out_shape=jax.ShapeDtypeStruct((N_NODES, H), jnp.float32),
    )(h, parts, dparts, jnp.ones((NW, 1), jnp.float32), h_attn,
      vec(gbias), vec(w1b), vec(b1b), vec(w2b),
      vec(b2b), mw1, vec(mb1), mw2, vec(mb2), vec(w3b), vec(b3b))


def _pool_body(starts_ref, counts_ref, h_ref, fw_ref, fb_ref, o_ref,
               pooled_ref):
    g = pl.program_id(0)
    start = starts_ref[g]
    count = counts_ref[g]
    hs = h_ref[pl.ds(start, MAX_LEN), :]
    row = lax.broadcasted_iota(jnp.int32, (MAX_LEN, H), 0)
    s = jnp.sum(jnp.where(row < count, hs, 0.0), axis=0, keepdims=True)
    denom = jnp.maximum(count, 1).astype(jnp.float32)
    pooled_ref[pl.ds(g, 1), :] = s / denom

    @pl.when(g == N_GRAPHS - 1)
    def _():
        logits = pooled_ref[...] @ fw_ref[...] + fb_ref[...]
        m = jnp.max(logits, axis=-1, keepdims=True)
        lse = jnp.log(jnp.sum(jnp.exp(logits - m), axis=-1, keepdims=True)) + m
        o_ref[...] = logits - lse


def _pool_logits(h_pad, starts, counts, fin_W, fin_b):
    return pl.pallas_call(
        _pool_body,
        grid=(N_GRAPHS,),
        in_specs=[
            pl.BlockSpec(memory_space=pltpu.SMEM),
            pl.BlockSpec(memory_space=pltpu.SMEM),
            pl.BlockSpec((_PAD, H), lambda g: (0, 0)),
            pl.BlockSpec((H, N_CLASSES), lambda g: (0, 0)),
            pl.BlockSpec((1, N_CLASSES), lambda g: (0, 0)),
        ],
        out_specs=pl.BlockSpec((N_GRAPHS, N_CLASSES), lambda g: (0, 0)),
        out_shape=jax.ShapeDtypeStruct((N_GRAPHS, N_CLASSES), jnp.float32),
        scratch_shapes=[pltpu.VMEM((N_GRAPHS, H), jnp.float32)],
    )(starts, counts, h_pad, fin_W, fin_b[None])


def kernel(x, edge_index, batch, params):
    n = x.shape[0]
    loops = jnp.arange(n, dtype=jnp.int32)
    pad = jnp.full((E_PAD - 320000 - n,), DUMMY, jnp.int32)
    src = jnp.concatenate([edge_index[0].astype(jnp.int32), loops, pad])
    dst = jnp.concatenate([edge_index[1].astype(jnp.int32), loops, pad])

    counts = jnp.bincount(batch, length=N_GRAPHS).astype(jnp.int32)
    starts = jnp.concatenate([jnp.zeros((1,), jnp.int32),
                              jnp.cumsum(counts)[:-1].astype(jnp.int32)])
    b_idx = batch.astype(jnp.int32)

    h = _pre_mlp(x, params['pre_W1'], params['pre_b1'],
                 params['pre_W2'], params['pre_b2'])
    for i in range(LAYERS):
        h_npad = jnp.pad(h, ((0, NPAD - N_NODES), (0, 0)))
        xl, xr = _gat_proj(h_npad, params[f'gat{i}_Wl'], params[f'gat{i}_bl'],
                           params[f'gat{i}_Wr'], params[f'gat{i}_br'])
        att = params[f'gat{i}_att']
        s_rows = _sc_stream(xl, xr, src, dst)
        alpha, g = _tc_alpha(s_rows, att)
        ex, dparts = _sc_den(alpha, dst, g)
        gparts = _sc_scatter(xl, src, dst, ex,
                             jnp.zeros((640, H), jnp.float32))

        h_pad = jnp.pad(h, ((0, MAX_LEN), (0, 0)))
        h_attn = _mha_pallas(h_pad, starts, counts,
                             params[f'attn{i}_Wqkv'], params[f'attn{i}_bqkv'],
                             params[f'attn{i}_Wo'], params[f'attn{i}_bo'])

        h = _combine(h, gparts, dparts, h_attn,
                     params[f'gat{i}_bias'],
                     params[f'bn{i}_1_w'], params[f'bn{i}_1_b'],
                     params[f'bn{i}_2_w'], params[f'bn{i}_2_b'],
                     params[f'mlp{i}_W1'], params[f'mlp{i}_b1'],
                     params[f'mlp{i}_W2'], params[f'mlp{i}_b2'],
                     params[f'bn{i}_3_w'], params[f'bn{i}_3_b'])

    h_pad = jnp.pad(h, ((0, MAX_LEN), (0, 0)))
    return _pool_logits(h_pad, starts, counts, params['fin_W'], params['fin_b'])


# speedup vs baseline: 1.0639x; 1.0639x over previous
import functools

import jax
import jax.numpy as jnp
import numpy as np
from jax import lax
from jax.experimental import pallas as pl
from jax.experimental.pallas import tpu as pltpu
from jax.experimental.pallas import tpu_sc as plsc

N_NODES = 10000
D_FEAT = 128
N_GRAPHS = 64
N_CLASSES = 10
H = 64
AH = 4
HD = H // AH
LAYERS = 3
BN_EPS = 1e-05
MAX_LEN = 512

_BLK = 2000
_PAD = N_NODES + MAX_LEN
NPAD = 10240
DUMMY = N_NODES

NW = 32
W_E = 10368
E_PAD = NW * W_E
CH = 576
NCH = W_E // CH
NT = CH // 16

_INV_SQRT1P = float(1.0 / np.sqrt(1.0 + BN_EPS))


def _gelu_exact(x):
    return 0.5 * x * (1.0 + lax.erf(x * float(1.0 / np.sqrt(2.0))))


def _pre_mlp_body(x_ref, w1_ref, b1_ref, w2_ref, b2_ref, o_ref):
    h = _gelu_exact(x_ref[...] @ w1_ref[...] + b1_ref[...])
    o_ref[...] = _gelu_exact(h @ w2_ref[...] + b2_ref[...])


def _pre_mlp(x, W1, b1, W2, b2):
    return pl.pallas_call(
        _pre_mlp_body,
        grid=(N_NODES // _BLK,),
        in_specs=[
            pl.BlockSpec((_BLK, D_FEAT), lambda i: (i, 0)),
            pl.BlockSpec((D_FEAT, 2 * H), lambda i: (0, 0)),
            pl.BlockSpec((1, 2 * H), lambda i: (0, 0)),
            pl.BlockSpec((2 * H, H), lambda i: (0, 0)),
            pl.BlockSpec((1, H), lambda i: (0, 0)),
        ],
        out_specs=pl.BlockSpec((_BLK, H), lambda i: (i, 0)),
        out_shape=jax.ShapeDtypeStruct((N_NODES, H), jnp.float32),
    )(x, W1, b1[None, :], W2, b2[None, :])


def _proj_body(h_ref, wl_ref, bl_ref, wr_ref, br_ref, xl_ref, xr_ref):
    h = h_ref[...]
    xl_ref[...] = h @ wl_ref[...] + bl_ref[...]
    xr_ref[...] = h @ wr_ref[...] + br_ref[...]


def _gat_proj(h_npad, Wl, bl, Wr, br):
    return pl.pallas_call(
        _proj_body,
        grid=(NPAD // 1280,),
        in_specs=[
            pl.BlockSpec((1280, H), lambda i: (i, 0)),
            pl.BlockSpec((H, H), lambda i: (0, 0)),
            pl.BlockSpec((1, H), lambda i: (0, 0)),
            pl.BlockSpec((H, H), lambda i: (0, 0)),
            pl.BlockSpec((1, H), lambda i: (0, 0)),
        ],
        out_specs=[
            pl.BlockSpec((1280, H), lambda i: (i, 0)),
            pl.BlockSpec((1280, H), lambda i: (i, 0)),
        ],
        out_shape=[jax.ShapeDtypeStruct((NPAD, H), jnp.float32),
                   jax.ShapeDtypeStruct((NPAD, H), jnp.float32)],
    )(h_npad, Wl, bl[None, :], Wr, br[None, :])


def _shift_body(xl_ref, xr_ref, absatt_ref, c_ref):
    aa = absatt_ref[...]
    P = jnp.abs(xl_ref[...]) @ aa
    Q = jnp.abs(xr_ref[...]) @ aa
    c_ref[...] = Q + jnp.max(P)


def _gat_shift(xl, xr, att):
    c = pl.pallas_call(
        _shift_body,
        in_specs=[
            pl.BlockSpec((NPAD, H), lambda: (0, 0)),
            pl.BlockSpec((NPAD, H), lambda: (0, 0)),
            pl.BlockSpec((H, 1), lambda: (0, 0)),
        ],
        out_specs=pl.BlockSpec((NPAD, 1), lambda: (0, 0)),
        out_shape=jax.ShapeDtypeStruct((NPAD, 1), jnp.float32),
    )(xl, xr, jnp.abs(att)[:, None])
    return c.reshape(NPAD)


_SC_MESH = plsc.VectorSubcoreMesh(core_axis_name="c", subcore_axis_name="s")
_SC_PARAMS = pltpu.CompilerParams(needs_layout_passes=False,
                                  use_tc_tiling_on_sc=False)


@functools.partial(
    pl.kernel, mesh=_SC_MESH,
    out_type=[jax.ShapeDtypeStruct((E_PAD,), jnp.float32),
              jax.ShapeDtypeStruct((NW, NPAD), jnp.float32)],
    scratch_types=[pltpu.VMEM((NPAD,), jnp.float32),
                   pltpu.VMEM((NPAD,), jnp.float32),
                   pltpu.VMEM((H,), jnp.float32),
                   pltpu.VMEM((CH,), jnp.int32),
                   pltpu.VMEM((CH,), jnp.int32),
                   pltpu.VMEM((CH, H), jnp.float32),
                   pltpu.VMEM((CH,), jnp.float32),
                   pltpu.SemaphoreType.DMA],
    compiler_params=_SC_PARAMS,
)
def _sc_alpha(xl_hbm, xr_hbm, src_hbm, dst_hbm, c_hbm, att_hbm,
              ex_hbm, dparts_hbm,
              c_v, den_v, att_v, src_v, dst_v, s_v, ex_v, sem1):
    cc = lax.axis_index("c")
    ss = lax.axis_index("s")
    wid = ss * 2 + cc
    lanes = lax.iota(jnp.int32, 16)
    zero16 = jnp.zeros((16,), jnp.float32)

    pltpu.sync_copy(att_hbm, att_v)
    pltpu.sync_copy(c_hbm, c_v)

    def zbody(i, _):
        den_v[pl.ds(i * 16, 16)] = zero16
        return 0

    lax.fori_loop(0, NPAD // 16, zbody, 0)

    def chunk(j, _):
        base = wid * W_E + j * CH
        pltpu.sync_copy(src_hbm.at[pl.ds(base, CH)], src_v)
        pltpu.sync_copy(dst_hbm.at[pl.ds(base, CH)], dst_v)
        pltpu.async_copy(xr_hbm.at[dst_v], s_v, sem1).wait()
        pltpu.async_copy(xl_hbm.at[src_v], s_v, sem1, add=True).wait()

        def group(t, _):
            rows = t * 16 + lanes
            acc0 = zero16
            acc1 = zero16
            acc2 = zero16
            acc3 = zero16
            for f16 in range(H // 16):
                av = att_v[pl.ds(f16 * 16, 16)]
                for j16 in range(16):
                    f = f16 * 16 + j16
                    fv = jnp.full((16,), f, jnp.int32)
                    s = plsc.load_gather(s_v, [rows, fv])
                    term = jnp.maximum(s, 0.2 * s) * av[j16]
                    if f % 4 == 0:
                        acc0 = acc0 + term
                    elif f % 4 == 1:
                        acc1 = acc1 + term
                    elif f % 4 == 2:
                        acc2 = acc2 + term
                    else:
                        acc3 = acc3 + term
            alpha = (acc0 + acc1) + (acc2 + acc3)
            dstv = dst_v[pl.ds(t * 16, 16)]
            cv = plsc.load_gather(c_v, [dstv])
            ex = jnp.exp(alpha - cv)
            ex_v[pl.ds(t * 16, 16)] = ex
            plsc.addupdate_scatter(den_v, [dstv], ex)
            return 0

        lax.fori_loop(0, NT, group, 0)
        pltpu.sync_copy(ex_v, ex_hbm.at[pl.ds(base, CH)])
        return 0

    lax.fori_loop(0, NCH, chunk, 0)
    pltpu.sync_copy(den_v, dparts_hbm.at[wid])


@functools.partial(
    pl.kernel, mesh=_SC_MESH,
    out_type=jax.ShapeDtypeStruct((2, NPAD, H), jnp.float32),
    scratch_types=[pltpu.VMEM((CH,), jnp.int32),
                   pltpu.VMEM((CH,), jnp.int32),
                   pltpu.VMEM((CH, H), jnp.float32),
                   pltpu.VMEM((CH,), jnp.float32),
                   pltpu.VMEM((CH,), jnp.int32),
                   pltpu.VMEM((CH,), jnp.int32),
                   pltpu.VMEM((CH, H), jnp.float32),
                   pltpu.VMEM((CH,), jnp.float32),
                   pltpu.VMEM_SHARED((NPAD, H), jnp.float32),
                   pltpu.SemaphoreType.DMA,
                   pltpu.SemaphoreType.DMA,
                   pltpu.SemaphoreType.DMA,
                   pltpu.SemaphoreType.DMA],
    compiler_params=_SC_PARAMS,
)
def _sc_scatter(xl_hbm, src_hbm, dst_hbm, ex_hbm, zeros_hbm,
                out_hbm,
                src0, dst0, el0, ex0, src1, dst1, el1, ex1,
                accum_sh, sg0, sg1, sc0, sc1):
    cc = lax.axis_index("c")
    ss = lax.axis_index("s")
    wid = ss * 2 + cc
    lanes = lax.iota(jnp.int32, 16)

    pltpu.sync_copy(zeros_hbm, accum_sh.at[pl.ds(ss * 640, 640)])
    plsc.subcore_barrier()

    def load_idx(j, srcv, dstv, exv):
        base = wid * W_E + j * CH
        pltpu.sync_copy(src_hbm.at[pl.ds(base, CH)], srcv)
        pltpu.sync_copy(dst_hbm.at[pl.ds(base, CH)], dstv)
        pltpu.sync_copy(ex_hbm.at[pl.ds(base, CH)], exv)

    def scale(elv, exv):
        def group(t, _):
            rows = t * 16 + lanes
            ex = exv[pl.ds(t * 16, 16)]
            for f in range(H):
                fv = jnp.full((16,), f, jnp.int32)
                val = plsc.load_gather(elv, [rows, fv]) * ex
                plsc.store_scatter(elv, [rows, fv], val)
            return 0

        lax.fori_loop(0, NT, group, 0)

    load_idx(0, src0, dst0, ex0)
    pltpu.async_copy(xl_hbm.at[src0], el0, sg0)

    def pair(jj, _):
        j = jj * 2
        pltpu.make_async_copy(xl_hbm.at[src0], el0, sg0).wait()
        scale(el0, ex0)

        @pl.when(jj > 0)
        def _():
            pltpu.make_async_copy(el1, accum_sh.at[dst1], sc1).wait()

        load_idx(j + 1, src1, dst1, ex1)
        pltpu.async_copy(xl_hbm.at[src1], el1, sg1)
        pltpu.async_copy(el0, accum_sh.at[dst0], sc0, add=True)
        pltpu.make_async_copy(xl_hbm.at[src1], el1, sg1).wait()
        scale(el1, ex1)
        pltpu.make_async_copy(el0, accum_sh.at[dst0], sc0).wait()

        @pl.when(jj < NCH // 2 - 1)
        def _():
            load_idx(j + 2, src0, dst0, ex0)
            pltpu.async_copy(xl_hbm.at[src0], el0, sg0)

        pltpu.async_copy(el1, accum_sh.at[dst1], sc1, add=True)
        return 0

    lax.fori_loop(0, NCH // 2, pair, 0)
    pltpu.make_async_copy(el1, accum_sh.at[dst1], sc1).wait()
    plsc.subcore_barrier()
    pltpu.sync_copy(accum_sh.at[pl.ds(ss * 640, 640)],
                    out_hbm.at[cc, pl.ds(ss * 640, 640)])


def _mha_body(starts_ref, counts_ref, h_ref, wqkv_ref, bqkv_ref, wo_ref,
              bo_ref, out_ref):
    g = pl.program_id(0)
    start = starts_ref[g]
    count = counts_ref[g]
    hs = h_ref[pl.ds(start, MAX_LEN), :]
    qkv = hs @ wqkv_ref[...] + bqkv_ref[...]
    col = lax.broadcasted_iota(jnp.int32, (MAX_LEN, MAX_LEN), 1)
    kmask = col < count
    os = []
    for a in range(AH):
        q = qkv[:, a * HD:(a + 1) * HD]
        k = qkv[:, H + a * HD:H + (a + 1) * HD]
        v = qkv[:, 2 * H + a * HD:2 * H + (a + 1) * HD]
        s = lax.dot_general(q, k, (((1,), (1,)), ((), ()))) * (1.0 / np.sqrt(HD))
        s = jnp.where(kmask, s, -1e9)
        m = jnp.max(s, axis=-1, keepdims=True)
        p = jnp.exp(s - m)
        denom = jnp.sum(p, axis=-1, keepdims=True)
        os.append((p / denom) @ v)
    o = jnp.concatenate(os, axis=-1) @ wo_ref[...] + bo_ref[...]
    row = lax.broadcasted_iota(jnp.int32, (MAX_LEN, H), 0)
    cur = out_ref[pl.ds(start, MAX_LEN), :]
    out_ref[pl.ds(start, MAX_LEN), :] = jnp.where(row < count, o, cur)


def _mha_pallas(h_pad, starts, counts, Wqkv, bqkv, Wo, bo):
    out = pl.pallas_call(
        _mha_body,
        grid=(N_GRAPHS,),
        in_specs=[
            pl.BlockSpec(memory_space=pltpu.SMEM),
            pl.BlockSpec(memory_space=pltpu.SMEM),
            pl.BlockSpec((_PAD, H), lambda g: (0, 0)),
            pl.BlockSpec((H, 3 * H), lambda g: (0, 0)),
            pl.BlockSpec((1, 3 * H), lambda g: (0, 0)),
            pl.BlockSpec((H, H), lambda g: (0, 0)),
            pl.BlockSpec((1, H), lambda g: (0, 0)),
        ],
        out_specs=pl.BlockSpec((_PAD, H), lambda g: (0, 0)),
        out_shape=jax.ShapeDtypeStruct((_PAD, H), jnp.float32),
    )(starts, counts, h_pad, Wqkv, bqkv[None], Wo, bo[None])
    return out[:N_NODES]


def _combine_body(h_ref, p_ref, dp_ref, ones_ref, hat_ref,
                  gbias_ref, w1b_ref, b1b_ref, w2b_ref, b2b_ref,
                  mw1_ref, mb1_ref, mw2_ref, mb2_ref, w3b_ref, b3b_ref,
                  o_ref):
    h = h_ref[...]
    den = lax.dot_general(dp_ref[...], ones_ref[...],
                          (((0,), (0,)), ((), ())))
    gat = (p_ref[0] + p_ref[1])[:N_NODES] / den[:N_NODES]
    gat = gat + gbias_ref[...]
    h_local = (gat + h) * _INV_SQRT1P * w1b_ref[...] + b1b_ref[...]
    h_attn = (hat_ref[...] + h) * _INV_SQRT1P * w2b_ref[...] + b2b_ref[...]
    out = h_local + h_attn
    m = jnp.maximum(out @ mw1_ref[...] + mb1_ref[...], 0.0)
    m = m @ mw2_ref[...] + mb2_ref[...]
    out = (out + m) * _INV_SQRT1P * w3b_ref[...] + b3b_ref[...]
    o_ref[...] = jnp.maximum(out, 0.0)


def _combine(h, parts, dparts, h_attn, gbias, w1b, b1b, w2b, b2b,
             mw1, mb1, mw2, mb2, w3b, b3b):
    vec = lambda a: a[None, :]
    full2 = lambda shape: pl.BlockSpec(shape, lambda: tuple(0 for _ in shape))
    return pl.pallas_call(
        _combine_body,
        in_specs=[
            full2((N_NODES, H)),
            full2((2, NPAD, H)),
            full2((NW, NPAD)),
            full2((NW, 1)),
            full2((N_NODES, H)),
            full2((1, H)),
            full2((1, H)),
            full2((1, H)),
            full2((1, H)),
            full2((1, H)),
            full2((H, 2 * H)),
            full2((1, 2 * H)),
            full2((2 * H, H)),
            full2((1, H)),
            full2((1, H)),
            full2((1, H)),
        ],
        out_specs=full2((N_NODES, H)),
        out_shape=jax.ShapeDtypeStruct((N_NODES, H), jnp.float32),
    )(h, parts, dparts, jnp.ones((NW, 1), jnp.float32), h_attn,
      vec(gbias), vec(w1b), vec(b1b), vec(w2b),
      vec(b2b), mw1, vec(mb1), mw2, vec(mb2), vec(w3b), vec(b3b))


def _pool_body(starts_ref, counts_ref, h_ref, fw_ref, fb_ref, o_ref,
               pooled_ref):
    g = pl.program_id(0)
    start = starts_ref[g]
    count = counts_ref[g]
    hs = h_ref[pl.ds(start, MAX_LEN), :]
    row = lax.broadcasted_iota(jnp.int32, (MAX_LEN, H), 0)
    s = jnp.sum(jnp.where(row < count, hs, 0.0), axis=0, keepdims=True)
    denom = jnp.maximum(count, 1).astype(jnp.float32)
    pooled_ref[pl.ds(g, 1), :] = s / denom

    @pl.when(g == N_GRAPHS - 1)
    def _():
        logits = pooled_ref[...] @ fw_ref[...] + fb_ref[...]
        m = jnp.max(logits, axis=-1, keepdims=True)
        lse = jnp.log(jnp.sum(jnp.exp(logits - m), axis=-1, keepdims=True)) + m
        o_ref[...] = logits - lse


def _pool_logits(h_pad, starts, counts, fin_W, fin_b):
    return pl.pallas_call(
        _pool_body,
        grid=(N_GRAPHS,),
        in_specs=[
            pl.BlockSpec(memory_space=pltpu.SMEM),
            pl.BlockSpec(memory_space=pltpu.SMEM),
            pl.BlockSpec((_PAD, H), lambda g: (0, 0)),
            pl.BlockSpec((H, N_CLASSES), lambda g: (0, 0)),
            pl.BlockSpec((1, N_CLASSES), lambda g: (0, 0)),
        ],
        out_specs=pl.BlockSpec((N_GRAPHS, N_CLASSES), lambda g: (0, 0)),
        out_shape=jax.ShapeDtypeStruct((N_GRAPHS, N_CLASSES), jnp.float32),
        scratch_shapes=[pltpu.VMEM((N_GRAPHS, H), jnp.float32)],
    )(starts, counts, h_pad, fin_W, fin_b[None])


def kernel(x, edge_index, batch, params):
    n = x.shape[0]
    loops = jnp.arange(n, dtype=jnp.int32)
    pad = jnp.full((E_PAD - 320000 - n,), DUMMY, jnp.int32)
    src = jnp.concatenate([edge_index[0].astype(jnp.int32), loops, pad])
    dst = jnp.concatenate([edge_index[1].astype(jnp.int32), loops, pad])

    counts = jnp.bincount(batch, length=N_GRAPHS).astype(jnp.int32)
    starts = jnp.concatenate([jnp.zeros((1,), jnp.int32),
                              jnp.cumsum(counts)[:-1].astype(jnp.int32)])
    b_idx = batch.astype(jnp.int32)

    h = _pre_mlp(x, params['pre_W1'], params['pre_b1'],
                 params['pre_W2'], params['pre_b2'])
    for i in range(LAYERS):
        h_npad = jnp.pad(h, ((0, NPAD - N_NODES), (0, 0)))
        xl, xr = _gat_proj(h_npad, params[f'gat{i}_Wl'], params[f'gat{i}_bl'],
                           params[f'gat{i}_Wr'], params[f'gat{i}_br'])
        att = params[f'gat{i}_att']
        c = _gat_shift(xl, xr, att)
        ex, dparts = _sc_alpha(xl, xr, src, dst, c, att)
        gparts = _sc_scatter(xl, src, dst, ex,
                             jnp.zeros((640, H), jnp.float32))

        h_pad = jnp.pad(h, ((0, MAX_LEN), (0, 0)))
        h_attn = _mha_pallas(h_pad, starts, counts,
                             params[f'attn{i}_Wqkv'], params[f'attn{i}_bqkv'],
                             params[f'attn{i}_Wo'], params[f'attn{i}_bo'])

        h = _combine(h, gparts, dparts, h_attn,
                     params[f'gat{i}_bias'],
                     params[f'bn{i}_1_w'], params[f'bn{i}_1_b'],
                     params[f'bn{i}_2_w'], params[f'bn{i}_2_b'],
                     params[f'mlp{i}_W1'], params[f'mlp{i}_b1'],
                     params[f'mlp{i}_W2'], params[f'mlp{i}_b2'],
                     params[f'bn{i}_3_w'], params[f'bn{i}_3_b'])

    h_pad = jnp.pad(h, ((0, MAX_LEN), (0, 0)))
    return _pool_logits(h_pad, starts, counts, params['fin_W'], params['fin_b'])


# speedup vs baseline: 1.1286x; 1.0608x over previous
import functools

import jax
import jax.numpy as jnp
import numpy as np
from jax import lax
from jax.experimental import pallas as pl
from jax.experimental.pallas import tpu as pltpu
from jax.experimental.pallas import tpu_sc as plsc

N_NODES = 10000
D_FEAT = 128
N_GRAPHS = 64
N_CLASSES = 10
H = 64
AH = 4
HD = H // AH
LAYERS = 3
BN_EPS = 1e-05
MAX_LEN = 512

_BLK = 2000
_PAD = N_NODES + MAX_LEN
NPAD = 10240
DUMMY = N_NODES

NW = 32
W_E = 10368
E_PAD = NW * W_E
CH = 576
NCH = W_E // CH
NT = CH // 16

_INV_SQRT1P = float(1.0 / np.sqrt(1.0 + BN_EPS))


def _gelu_exact(x):
    return 0.5 * x * (1.0 + lax.erf(x * float(1.0 / np.sqrt(2.0))))


def _pre_mlp_body(x_ref, w1_ref, b1_ref, w2_ref, b2_ref, o_ref):
    h = _gelu_exact(x_ref[...] @ w1_ref[...] + b1_ref[...])
    o_ref[...] = _gelu_exact(h @ w2_ref[...] + b2_ref[...])


def _pre_mlp(x, W1, b1, W2, b2):
    return pl.pallas_call(
        _pre_mlp_body,
        grid=(N_NODES // _BLK,),
        in_specs=[
            pl.BlockSpec((_BLK, D_FEAT), lambda i: (i, 0)),
            pl.BlockSpec((D_FEAT, 2 * H), lambda i: (0, 0)),
            pl.BlockSpec((1, 2 * H), lambda i: (0, 0)),
            pl.BlockSpec((2 * H, H), lambda i: (0, 0)),
            pl.BlockSpec((1, H), lambda i: (0, 0)),
        ],
        out_specs=pl.BlockSpec((_BLK, H), lambda i: (i, 0)),
        out_shape=jax.ShapeDtypeStruct((N_NODES, H), jnp.float32),
    )(x, W1, b1[None, :], W2, b2[None, :])


def _proj_body(h_ref, wl_ref, bl_ref, wr_ref, br_ref, xl_ref, xr_ref):
    h = h_ref[...]
    xl_ref[...] = h @ wl_ref[...] + bl_ref[...]
    xr_ref[...] = h @ wr_ref[...] + br_ref[...]


def _gat_proj(h_npad, Wl, bl, Wr, br):
    return pl.pallas_call(
        _proj_body,
        grid=(NPAD // 1280,),
        in_specs=[
            pl.BlockSpec((1280, H), lambda i: (i, 0)),
            pl.BlockSpec((H, H), lambda i: (0, 0)),
            pl.BlockSpec((1, H), lambda i: (0, 0)),
            pl.BlockSpec((H, H), lambda i: (0, 0)),
            pl.BlockSpec((1, H), lambda i: (0, 0)),
        ],
        out_specs=[
            pl.BlockSpec((1280, H), lambda i: (i, 0)),
            pl.BlockSpec((1280, H), lambda i: (i, 0)),
        ],
        out_shape=[jax.ShapeDtypeStruct((NPAD, H), jnp.float32),
                   jax.ShapeDtypeStruct((NPAD, H), jnp.float32)],
    )(h_npad, Wl, bl[None, :], Wr, br[None, :])


def _shift_body(xl_ref, xr_ref, absatt_ref, c_ref):
    aa = absatt_ref[...]
    P = jnp.abs(xl_ref[...]) @ aa
    Q = jnp.abs(xr_ref[...]) @ aa
    c_ref[...] = Q + jnp.max(P)


def _gat_shift(xl, xr, att):
    c = pl.pallas_call(
        _shift_body,
        in_specs=[
            pl.BlockSpec((NPAD, H), lambda: (0, 0)),
            pl.BlockSpec((NPAD, H), lambda: (0, 0)),
            pl.BlockSpec((H, 1), lambda: (0, 0)),
        ],
        out_specs=pl.BlockSpec((NPAD, 1), lambda: (0, 0)),
        out_shape=jax.ShapeDtypeStruct((NPAD, 1), jnp.float32),
    )(xl, xr, jnp.abs(att)[:, None])
    return c.reshape(NPAD)


_SC_MESH = plsc.VectorSubcoreMesh(core_axis_name="c", subcore_axis_name="s")
_SC_PARAMS = pltpu.CompilerParams(needs_layout_passes=False,
                                  use_tc_tiling_on_sc=False)


@functools.partial(
    pl.kernel, mesh=_SC_MESH,
    out_type=[jax.ShapeDtypeStruct((E_PAD,), jnp.float32),
              jax.ShapeDtypeStruct((NW, NPAD), jnp.float32)],
    scratch_types=[pltpu.VMEM((NPAD,), jnp.float32),
                   pltpu.VMEM((NPAD,), jnp.float32),
                   pltpu.VMEM((H,), jnp.float32),
                   pltpu.VMEM((CH,), jnp.int32),
                   pltpu.VMEM((CH,), jnp.int32),
                   pltpu.VMEM((CH, H), jnp.float32),
                   pltpu.VMEM((CH,), jnp.int32),
                   pltpu.VMEM((CH,), jnp.int32),
                   pltpu.VMEM((CH, H), jnp.float32),
                   pltpu.VMEM((CH,), jnp.float32),
                   pltpu.SemaphoreType.DMA,
                   pltpu.SemaphoreType.DMA],
    compiler_params=_SC_PARAMS,
)
def _sc_alpha(xl_hbm, xr_hbm, src_hbm, dst_hbm, c_hbm, att_hbm,
              ex_hbm, dparts_hbm,
              c_v, den_v, att_v, src0, dst0, s0, src1, dst1, s1, ex_v,
              sa0, sa1):
    cc = lax.axis_index("c")
    ss = lax.axis_index("s")
    wid = ss * 2 + cc
    lanes = lax.iota(jnp.int32, 16)
    zero16 = jnp.zeros((16,), jnp.float32)

    pltpu.sync_copy(att_hbm, att_v)
    pltpu.sync_copy(c_hbm, c_v)

    def zbody(i, _):
        den_v[pl.ds(i * 16, 16)] = zero16
        return 0

    lax.fori_loop(0, NPAD // 16, zbody, 0)

    def load_idx(j, srcv, dstv):
        base = wid * W_E + j * CH
        pltpu.sync_copy(src_hbm.at[pl.ds(base, CH)], srcv)
        pltpu.sync_copy(dst_hbm.at[pl.ds(base, CH)], dstv)

    def compute(j, s_v, dst_v):
        def group(t, _):
            rows = t * 16 + lanes
            acc0 = zero16
            acc1 = zero16
            acc2 = zero16
            acc3 = zero16
            for f16 in range(H // 16):
                av = att_v[pl.ds(f16 * 16, 16)]
                for j16 in range(16):
                    f = f16 * 16 + j16
                    fv = jnp.full((16,), f, jnp.int32)
                    s = plsc.load_gather(s_v, [rows, fv])
                    term = jnp.maximum(s, 0.2 * s) * av[j16]
                    if f % 4 == 0:
                        acc0 = acc0 + term
                    elif f % 4 == 1:
                        acc1 = acc1 + term
                    elif f % 4 == 2:
                        acc2 = acc2 + term
                    else:
                        acc3 = acc3 + term
            alpha = (acc0 + acc1) + (acc2 + acc3)
            dstv = dst_v[pl.ds(t * 16, 16)]
            cv = plsc.load_gather(c_v, [dstv])
            ex = jnp.exp(alpha - cv)
            ex_v[pl.ds(t * 16, 16)] = ex
            plsc.addupdate_scatter(den_v, [dstv], ex)
            return 0

        lax.fori_loop(0, NT, group, 0)
        base = wid * W_E + j * CH
        pltpu.sync_copy(ex_v, ex_hbm.at[pl.ds(base, CH)])

    load_idx(0, src0, dst0)
    pltpu.async_copy(xr_hbm.at[dst0], s0, sa0)

    def pair(jj, _):
        j = jj * 2
        pltpu.make_async_copy(xr_hbm.at[dst0], s0, sa0).wait()
        pltpu.async_copy(xl_hbm.at[src0], s0, sa0, add=True)
        load_idx(j + 1, src1, dst1)
        pltpu.async_copy(xr_hbm.at[dst1], s1, sa1)
        pltpu.make_async_copy(xl_hbm.at[src0], s0, sa0).wait()
        compute(j, s0, dst0)
        pltpu.make_async_copy(xr_hbm.at[dst1], s1, sa1).wait()
        pltpu.async_copy(xl_hbm.at[src1], s1, sa1, add=True)

        @pl.when(jj < NCH // 2 - 1)
        def _():
            load_idx(j + 2, src0, dst0)
            pltpu.async_copy(xr_hbm.at[dst0], s0, sa0)

        pltpu.make_async_copy(xl_hbm.at[src1], s1, sa1).wait()
        compute(j + 1, s1, dst1)
        return 0

    lax.fori_loop(0, NCH // 2, pair, 0)
    pltpu.sync_copy(den_v, dparts_hbm.at[wid])


@functools.partial(
    pl.kernel, mesh=_SC_MESH,
    out_type=jax.ShapeDtypeStruct((2, NPAD, H), jnp.float32),
    scratch_types=[pltpu.VMEM((CH,), jnp.int32),
                   pltpu.VMEM((CH,), jnp.int32),
                   pltpu.VMEM((CH, H), jnp.float32),
                   pltpu.VMEM((CH,), jnp.float32),
                   pltpu.VMEM((CH,), jnp.int32),
                   pltpu.VMEM((CH,), jnp.int32),
                   pltpu.VMEM((CH, H), jnp.float32),
                   pltpu.VMEM((CH,), jnp.float32),
                   pltpu.VMEM_SHARED((NPAD, H), jnp.float32),
                   pltpu.SemaphoreType.DMA,
                   pltpu.SemaphoreType.DMA,
                   pltpu.SemaphoreType.DMA,
                   pltpu.SemaphoreType.DMA],
    compiler_params=_SC_PARAMS,
)
def _sc_scatter(xl_hbm, src_hbm, dst_hbm, ex_hbm, zeros_hbm,
                out_hbm,
                src0, dst0, el0, ex0, src1, dst1, el1, ex1,
                accum_sh, sg0, sg1, sc0, sc1):
    cc = lax.axis_index("c")
    ss = lax.axis_index("s")
    wid = ss * 2 + cc
    lanes = lax.iota(jnp.int32, 16)

    pltpu.sync_copy(zeros_hbm, accum_sh.at[pl.ds(ss * 640, 640)])
    plsc.subcore_barrier()

    def load_idx(j, srcv, dstv, exv):
        base = wid * W_E + j * CH
        pltpu.sync_copy(src_hbm.at[pl.ds(base, CH)], srcv)
        pltpu.sync_copy(dst_hbm.at[pl.ds(base, CH)], dstv)
        pltpu.sync_copy(ex_hbm.at[pl.ds(base, CH)], exv)

    def scale(elv, exv):
        def group(t, _):
            rows = t * 16 + lanes
            ex = exv[pl.ds(t * 16, 16)]
            for f in range(H):
                fv = jnp.full((16,), f, jnp.int32)
                val = plsc.load_gather(elv, [rows, fv]) * ex
                plsc.store_scatter(elv, [rows, fv], val)
            return 0

        lax.fori_loop(0, NT, group, 0)

    load_idx(0, src0, dst0, ex0)
    pltpu.async_copy(xl_hbm.at[src0], el0, sg0)

    def pair(jj, _):
        j = jj * 2
        pltpu.make_async_copy(xl_hbm.at[src0], el0, sg0).wait()
        scale(el0, ex0)

        @pl.when(jj > 0)
        def _():
            pltpu.make_async_copy(el1, accum_sh.at[dst1], sc1).wait()

        load_idx(j + 1, src1, dst1, ex1)
        pltpu.async_copy(xl_hbm.at[src1], el1, sg1)
        pltpu.async_copy(el0, accum_sh.at[dst0], sc0, add=True)
        pltpu.make_async_copy(xl_hbm.at[src1], el1, sg1).wait()
        scale(el1, ex1)
        pltpu.make_async_copy(el0, accum_sh.at[dst0], sc0).wait()

        @pl.when(jj < NCH // 2 - 1)
        def _():
            load_idx(j + 2, src0, dst0, ex0)
            pltpu.async_copy(xl_hbm.at[src0], el0, sg0)

        pltpu.async_copy(el1, accum_sh.at[dst1], sc1, add=True)
        return 0

    lax.fori_loop(0, NCH // 2, pair, 0)
    pltpu.make_async_copy(el1, accum_sh.at[dst1], sc1).wait()
    plsc.subcore_barrier()
    pltpu.sync_copy(accum_sh.at[pl.ds(ss * 640, 640)],
                    out_hbm.at[cc, pl.ds(ss * 640, 640)])


def _mha_body(starts_ref, counts_ref, h_ref, wqkv_ref, bqkv_ref, wo_ref,
              bo_ref, out_ref):
    g = pl.program_id(0)
    start = starts_ref[g]
    count = counts_ref[g]
    hs = h_ref[pl.ds(start, MAX_LEN), :]
    qkv = hs @ wqkv_ref[...] + bqkv_ref[...]
    col = lax.broadcasted_iota(jnp.int32, (MAX_LEN, MAX_LEN), 1)
    kmask = col < count
    os = []
    for a in range(AH):
        q = qkv[:, a * HD:(a + 1) * HD]
        k = qkv[:, H + a * HD:H + (a + 1) * HD]
        v = qkv[:, 2 * H + a * HD:2 * H + (a + 1) * HD]
        s = lax.dot_general(q, k, (((1,), (1,)), ((), ()))) * (1.0 / np.sqrt(HD))
        s = jnp.where(kmask, s, -1e9)
        m = jnp.max(s, axis=-1, keepdims=True)
        p = jnp.exp(s - m)
        denom = jnp.sum(p, axis=-1, keepdims=True)
        os.append((p / denom) @ v)
    o = jnp.concatenate(os, axis=-1) @ wo_ref[...] + bo_ref[...]
    row = lax.broadcasted_iota(jnp.int32, (MAX_LEN, H), 0)
    cur = out_ref[pl.ds(start, MAX_LEN), :]
    out_ref[pl.ds(start, MAX_LEN), :] = jnp.where(row < count, o, cur)


def _mha_pallas(h_pad, starts, counts, Wqkv, bqkv, Wo, bo):
    out = pl.pallas_call(
        _mha_body,
        grid=(N_GRAPHS,),
        in_specs=[
            pl.BlockSpec(memory_space=pltpu.SMEM),
            pl.BlockSpec(memory_space=pltpu.SMEM),
            pl.BlockSpec((_PAD, H), lambda g: (0, 0)),
            pl.BlockSpec((H, 3 * H), lambda g: (0, 0)),
            pl.BlockSpec((1, 3 * H), lambda g: (0, 0)),
            pl.BlockSpec((H, H), lambda g: (0, 0)),
            pl.BlockSpec((1, H), lambda g: (0, 0)),
        ],
        out_specs=pl.BlockSpec((_PAD, H), lambda g: (0, 0)),
        out_shape=jax.ShapeDtypeStruct((_PAD, H), jnp.float32),
    )(starts, counts, h_pad, Wqkv, bqkv[None], Wo, bo[None])
    return out[:N_NODES]


def _combine_body(h_ref, p_ref, dp_ref, ones_ref, hat_ref,
                  gbias_ref, w1b_ref, b1b_ref, w2b_ref, b2b_ref,
                  mw1_ref, mb1_ref, mw2_ref, mb2_ref, w3b_ref, b3b_ref,
                  o_ref):
    h = h_ref[...]
    den = lax.dot_general(dp_ref[...], ones_ref[...],
                          (((0,), (0,)), ((), ())))
    gat = (p_ref[0] + p_ref[1])[:N_NODES] / den[:N_NODES]
    gat = gat + gbias_ref[...]
    h_local = (gat + h) * _INV_SQRT1P * w1b_ref[...] + b1b_ref[...]
    h_attn = (hat_ref[...] + h) * _INV_SQRT1P * w2b_ref[...] + b2b_ref[...]
    out = h_local + h_attn
    m = jnp.maximum(out @ mw1_ref[...] + mb1_ref[...], 0.0)
    m = m @ mw2_ref[...] + mb2_ref[...]
    out = (out + m) * _INV_SQRT1P * w3b_ref[...] + b3b_ref[...]
    o_ref[...] = jnp.maximum(out, 0.0)


def _combine(h, parts, dparts, h_attn, gbias, w1b, b1b, w2b, b2b,
             mw1, mb1, mw2, mb2, w3b, b3b):
    vec = lambda a: a[None, :]
    full2 = lambda shape: pl.BlockSpec(shape, lambda: tuple(0 for _ in shape))
    return pl.pallas_call(
        _combine_body,
        in_specs=[
            full2((N_NODES, H)),
            full2((2, NPAD, H)),
            full2((NW, NPAD)),
            full2((NW, 1)),
            full2((N_NODES, H)),
            full2((1, H)),
            full2((1, H)),
            full2((1, H)),
            full2((1, H)),
            full2((1, H)),
            full2((H, 2 * H)),
            full2((1, 2 * H)),
            full2((2 * H, H)),
            full2((1, H)),
            full2((1, H)),
            full2((1, H)),
        ],
        out_specs=full2((N_NODES, H)),
        out_shape=jax.ShapeDtypeStruct((N_NODES, H), jnp.float32),
    )(h, parts, dparts, jnp.ones((NW, 1), jnp.float32), h_attn,
      vec(gbias), vec(w1b), vec(b1b), vec(w2b),
      vec(b2b), mw1, vec(mb1), mw2, vec(mb2), vec(w3b), vec(b3b))


def _pool_body(starts_ref, counts_ref, h_ref, fw_ref, fb_ref, o_ref,
               pooled_ref):
    g = pl.program_id(0)
    start = starts_ref[g]
    count = counts_ref[g]
    hs = h_ref[pl.ds(start, MAX_LEN), :]
    row = lax.broadcasted_iota(jnp.int32, (MAX_LEN, H), 0)
    s = jnp.sum(jnp.where(row < count, hs, 0.0), axis=0, keepdims=True)
    denom = jnp.maximum(count, 1).astype(jnp.float32)
    pooled_ref[pl.ds(g, 1), :] = s / denom

    @pl.when(g == N_GRAPHS - 1)
    def _():
        logits = pooled_ref[...] @ fw_ref[...] + fb_ref[...]
        m = jnp.max(logits, axis=-1, keepdims=True)
        lse = jnp.log(jnp.sum(jnp.exp(logits - m), axis=-1, keepdims=True)) + m
        o_ref[...] = logits - lse


def _pool_logits(h_pad, starts, counts, fin_W, fin_b):
    return pl.pallas_call(
        _pool_body,
        grid=(N_GRAPHS,),
        in_specs=[
            pl.BlockSpec(memory_space=pltpu.SMEM),
            pl.BlockSpec(memory_space=pltpu.SMEM),
            pl.BlockSpec((_PAD, H), lambda g: (0, 0)),
            pl.BlockSpec((H, N_CLASSES), lambda g: (0, 0)),
            pl.BlockSpec((1, N_CLASSES), lambda g: (0, 0)),
        ],
        out_specs=pl.BlockSpec((N_GRAPHS, N_CLASSES), lambda g: (0, 0)),
        out_shape=jax.ShapeDtypeStruct((N_GRAPHS, N_CLASSES), jnp.float32),
        scratch_shapes=[pltpu.VMEM((N_GRAPHS, H), jnp.float32)],
    )(starts, counts, h_pad, fin_W, fin_b[None])


def kernel(x, edge_index, batch, params):
    n = x.shape[0]
    loops = jnp.arange(n, dtype=jnp.int32)
    pad = jnp.full((E_PAD - 320000 - n,), DUMMY, jnp.int32)
    src = jnp.concatenate([edge_index[0].astype(jnp.int32), loops, pad])
    dst = jnp.concatenate([edge_index[1].astype(jnp.int32), loops, pad])

    counts = jnp.bincount(batch, length=N_GRAPHS).astype(jnp.int32)
    starts = jnp.concatenate([jnp.zeros((1,), jnp.int32),
                              jnp.cumsum(counts)[:-1].astype(jnp.int32)])
    b_idx = batch.astype(jnp.int32)

    h = _pre_mlp(x, params['pre_W1'], params['pre_b1'],
                 params['pre_W2'], params['pre_b2'])
    for i in range(LAYERS):
        h_npad = jnp.pad(h, ((0, NPAD - N_NODES), (0, 0)))
        xl, xr = _gat_proj(h_npad, params[f'gat{i}_Wl'], params[f'gat{i}_bl'],
                           params[f'gat{i}_Wr'], params[f'gat{i}_br'])
        att = params[f'gat{i}_att']
        c = _gat_shift(xl, xr, att)
        ex, dparts = _sc_alpha(xl, xr, src, dst, c, att)
        gparts = _sc_scatter(xl, src, dst, ex,
                             jnp.zeros((640, H), jnp.float32))

        h_pad = jnp.pad(h, ((0, MAX_LEN), (0, 0)))
        h_attn = _mha_pallas(h_pad, starts, counts,
                             params[f'attn{i}_Wqkv'], params[f'attn{i}_bqkv'],
                             params[f'attn{i}_Wo'], params[f'attn{i}_bo'])

        h = _combine(h, gparts, dparts, h_attn,
                     params[f'gat{i}_bias'],
                     params[f'bn{i}_1_w'], params[f'bn{i}_1_b'],
                     params[f'bn{i}_2_w'], params[f'bn{i}_2_b'],
                     params[f'mlp{i}_W1'], params[f'mlp{i}_b1'],
                     params[f'mlp{i}_W2'], params[f'mlp{i}_b2'],
                     params[f'bn{i}_3_w'], params[f'bn{i}_3_b'])

    h_pad = jnp.pad(h, ((0, MAX_LEN), (0, 0)))
    return _pool_logits(h_pad, starts, counts, params['fin_W'], params['fin_b'])
